# Initial kernel scaffold; baseline (speedup 1.0000x reference)
import functools

import jax
import jax.numpy as jnp
from jax import lax
from jax.experimental import pallas as pl
from jax.experimental.pallas import tpu as pltpu
from jax.experimental.pallas import tpu_sc as plsc

B, Q, S = 16, 2048, 512
L = 16


def _build(b, q, s, num_cores, num_subcores, chunk_q, interpret=False):
    nw = num_cores * num_subcores
    total_q = b * q
    qpw = total_q // nw
    n_chunks = qpw // chunk_q
    n_groups = chunk_q // L
    sv = s // L
    assert qpw % chunk_q == 0 and chunk_q % L == 0 and s % L == 0
    assert q % qpw == 0

    mesh = plsc.VectorSubcoreMesh(core_axis_name="c", subcore_axis_name="s")

    @functools.partial(
        pl.kernel,
        out_type=jax.ShapeDtypeStruct((total_q,), jnp.float32),
        mesh=mesh,
        scratch_types=[
            pltpu.VMEM((s,), jnp.float32),
            pltpu.VMEM((chunk_q * s,), jnp.float32),
            pltpu.VMEM((qpw,), jnp.float32),
        ],
        interpret=interpret,
    )
    def k(tw_hbm, occ_hbm, out_hbm, tw_v, occ_v, out_v):
        cid = lax.axis_index("c")
        sid = lax.axis_index("s")
        wid = sid * num_cores + cid
        prob = (wid * qpw) // q
        pltpu.sync_copy(tw_hbm.at[pl.ds(prob * s, s)], tw_v)

        def chunk_body(ci, carry):
            occ_off = wid * (qpw * s) + ci * (chunk_q * s)
            pltpu.sync_copy(occ_hbm.at[pl.ds(occ_off, chunk_q * s)], occ_v)

            def group_body(g, carry2):
                res = jnp.zeros((L,), jnp.float32)
                for qq in range(L):
                    qbase = (g * L + qq) * s
                    acc = occ_v[pl.ds(qbase, L)] * tw_v[pl.ds(0, L)]
                    for j in range(1, sv):
                        acc = acc + (occ_v[pl.ds(qbase + j * L, L)]
                                     * tw_v[pl.ds(j * L, L)])
                    tot = jnp.sum(acc)
                    res = jnp.where(lax.iota(jnp.int32, L) == qq, tot, res)
                out_v[pl.ds(ci * chunk_q + g * L, L)] = res
                return carry2

            lax.fori_loop(0, n_groups, group_body, 0)
            return carry

        lax.fori_loop(0, n_chunks, chunk_body, 0)
        pltpu.sync_copy(out_v, out_hbm.at[pl.ds(wid * qpw, qpw)])

    return k


def kernel(token_weight_flat, occurrence_flat, valid, symbol_row_splits,
           question_row_splits):
    del symbol_row_splits, question_row_splits
    tw_masked = token_weight_flat * jnp.repeat(valid, S).astype(jnp.float32)
    k = _build(B, Q, S, 2, 16, 64)
    logits = k(tw_masked, occurrence_flat)
    return logits, valid


# speedup vs baseline: 1845.2353x; 1845.2353x over previous
import functools

import jax
import jax.numpy as jnp
from jax import lax
from jax.experimental import pallas as pl
from jax.experimental.pallas import tpu as pltpu
from jax.experimental.pallas import tpu_sc as plsc

B, Q, S = 16, 2048, 512
L = 16


def _build(b, q, s, num_cores, num_subcores, chunk_q, interpret=False):
    nw = num_cores * num_subcores
    total_q = b * q
    qpw = total_q // nw
    n_chunks = qpw // chunk_q
    n_groups = chunk_q // L
    sv = s // L
    assert qpw % chunk_q == 0 and chunk_q % L == 0 and s % L == 0
    assert q % qpw == 0

    mesh = plsc.VectorSubcoreMesh(core_axis_name="c", subcore_axis_name="s",
                                  num_cores=num_cores,
                                  num_subcores=num_subcores)

    @functools.partial(
        pl.kernel,
        out_type=jax.ShapeDtypeStruct((total_q,), jnp.float32),
        mesh=mesh,
        scratch_types=[
            pltpu.VMEM((s,), jnp.float32),
            pltpu.VMEM((chunk_q * s,), jnp.float32),
            pltpu.VMEM((qpw,), jnp.float32),
            pltpu.VMEM((L * L,), jnp.float32),
        ],
        compiler_params=pltpu.CompilerParams(needs_layout_passes=False),
        interpret=interpret,
    )
    def k(tw_hbm, occ_hbm, out_hbm, tw_v, occ_v, out_v, tbuf):
        cid = lax.axis_index("c")
        sid = lax.axis_index("s")
        wid = sid * num_cores + cid
        prob = (wid * qpw) // q
        pltpu.sync_copy(tw_hbm.at[pl.ds(prob * s, s)], tw_v)

        def chunk_body(ci, carry):
            occ_off = wid * (qpw * s) + ci * (chunk_q * s)
            pltpu.sync_copy(occ_hbm.at[pl.ds(occ_off, chunk_q * s)], occ_v)

            def group_body(g, carry2):
                for qq in range(L):
                    qbase = (g * L + qq) * s
                    acc = occ_v[pl.ds(qbase, L)] * tw_v[pl.ds(0, L)]
                    for j in range(1, sv):
                        acc = acc + (occ_v[pl.ds(qbase + j * L, L)]
                                     * tw_v[pl.ds(j * L, L)])
                    tbuf[pl.ds(qq * L, L)] = acc
                col = lax.iota(jnp.int32, L) * L
                res = plsc.load_gather(tbuf, [col])
                for c in range(1, L):
                    res = res + plsc.load_gather(tbuf, [col + c])
                out_v[pl.ds(ci * chunk_q + g * L, L)] = res
                return carry2

            lax.fori_loop(0, n_groups, group_body, 0)
            return carry

        lax.fori_loop(0, n_chunks, chunk_body, 0)
        pltpu.sync_copy(out_v, out_hbm.at[pl.ds(wid * qpw, qpw)])

    return k


def kernel(token_weight_flat, occurrence_flat, valid, symbol_row_splits,
           question_row_splits):
    del symbol_row_splits, question_row_splits
    tw_masked = token_weight_flat * jnp.repeat(valid, S).astype(jnp.float32)
    k = _build(B, Q, S, 2, 16, 64)
    logits = k(tw_masked, occurrence_flat)
    return logits, valid


# speedup vs baseline: 2331.7396x; 1.2637x over previous
import functools

import jax
import jax.numpy as jnp
from jax import lax
from jax.experimental import pallas as pl
from jax.experimental.pallas import tpu as pltpu
from jax.experimental.pallas import tpu_sc as plsc

B, Q, S = 16, 2048, 512
L = 16


def _build(b, q, s, num_cores, num_subcores, chunk_q, interpret=False):
    nw = num_cores * num_subcores
    total_q = b * q
    qpw = total_q // nw
    n_chunks = qpw // chunk_q
    n_groups = chunk_q // L
    sv = s // L
    assert qpw % chunk_q == 0 and chunk_q % L == 0 and s % L == 0
    assert q % qpw == 0

    mesh = plsc.VectorSubcoreMesh(core_axis_name="c", subcore_axis_name="s",
                                  num_cores=num_cores,
                                  num_subcores=num_subcores)

    @functools.partial(
        pl.kernel,
        out_type=jax.ShapeDtypeStruct((total_q,), jnp.float32),
        mesh=mesh,
        scratch_types=[
            pltpu.VMEM((s,), jnp.float32),
            pltpu.VMEM((chunk_q * s,), jnp.float32),
            pltpu.VMEM((chunk_q * s,), jnp.float32),
            pltpu.VMEM((qpw,), jnp.float32),
            pltpu.VMEM((L * L,), jnp.float32),
            pltpu.SemaphoreType.DMA,
            pltpu.SemaphoreType.DMA,
        ],
        compiler_params=pltpu.CompilerParams(needs_layout_passes=False),
        interpret=interpret,
    )
    def k(tw_hbm, occ_hbm, out_hbm, tw_v, occ_a, occ_b, out_v, tbuf,
          sem_a, sem_b):
        cid = lax.axis_index("c")
        sid = lax.axis_index("s")
        wid = sid * num_cores + cid
        prob = (wid * qpw) // q
        occ_base = wid * (qpw * s)
        pltpu.sync_copy(tw_hbm.at[pl.ds(prob * s, s)], tw_v)

        def start_fetch(ci, buf, sem):
            off = occ_base + ci * (chunk_q * s)
            pltpu.make_async_copy(
                occ_hbm.at[pl.ds(off, chunk_q * s)], buf, sem).start()

        def compute_chunk(ci, occ_v):
            def group_body(g, carry2):
                for qq in range(L):
                    qbase = (g * L + qq) * s
                    accs = []
                    for k4 in range(4):
                        a = (occ_v[pl.ds(qbase + k4 * L, L)]
                             * tw_v[pl.ds(k4 * L, L)])
                        for j in range(k4 + 4, sv, 4):
                            a = a + (occ_v[pl.ds(qbase + j * L, L)]
                                     * tw_v[pl.ds(j * L, L)])
                        accs.append(a)
                    acc = (accs[0] + accs[1]) + (accs[2] + accs[3])
                    tbuf[pl.ds(qq * L, L)] = acc
                col = lax.iota(jnp.int32, L) * L
                res = plsc.load_gather(tbuf, [col])
                for c in range(1, L):
                    res = res + plsc.load_gather(tbuf, [col + c])
                out_v[pl.ds(ci * chunk_q + g * L, L)] = res
                return carry2

            lax.fori_loop(0, n_groups, group_body, 0)

        start_fetch(0, occ_a, sem_a)

        def pair_body(p, carry):
            for par, (buf, sem, obuf, osem) in enumerate(
                    ((occ_a, sem_a, occ_b, sem_b),
                     (occ_b, sem_b, occ_a, sem_a))):
                ci = p * 2 + par

                @pl.when(ci + 1 < n_chunks)
                def _():
                    start_fetch(ci + 1, obuf, osem)

                pltpu.make_async_copy(
                    occ_hbm.at[pl.ds(occ_base + ci * (chunk_q * s),
                                     chunk_q * s)], buf, sem).wait()
                compute_chunk(ci, buf)
            return carry

        lax.fori_loop(0, n_chunks // 2, pair_body, 0)
        pltpu.sync_copy(out_v, out_hbm.at[pl.ds(wid * qpw, qpw)])

    return k


def kernel(token_weight_flat, occurrence_flat, valid, symbol_row_splits,
           question_row_splits):
    del symbol_row_splits, question_row_splits
    tw_masked = token_weight_flat * jnp.repeat(valid, S).astype(jnp.float32)
    k = _build(B, Q, S, 2, 16, 64)
    logits = k(tw_masked, occurrence_flat)
    return logits, valid


# speedup vs baseline: 3134.6742x; 1.3444x over previous
import functools

import jax
import jax.numpy as jnp
from jax import lax
from jax.experimental import pallas as pl
from jax.experimental.pallas import tpu as pltpu
from jax.experimental.pallas import tpu_sc as plsc

B, Q, S = 16, 2048, 512
L = 16


def _build(b, q, s, num_cores, num_subcores, chunk_q, interpret=False):
    nw = num_cores * num_subcores
    total_q = b * q
    qpw = total_q // nw
    n_chunks = qpw // chunk_q
    n_groups = chunk_q // L
    sv = s // L
    assert qpw % chunk_q == 0 and chunk_q % L == 0 and s % L == 0
    assert q % qpw == 0

    mesh = plsc.VectorSubcoreMesh(core_axis_name="c", subcore_axis_name="s",
                                  num_cores=num_cores,
                                  num_subcores=num_subcores)

    @functools.partial(
        pl.kernel,
        out_type=jax.ShapeDtypeStruct((total_q,), jnp.float32),
        mesh=mesh,
        scratch_types=[
            pltpu.VMEM((s,), jnp.float32),
            pltpu.VMEM((chunk_q * s,), jnp.float32),
            pltpu.VMEM((chunk_q * s,), jnp.float32),
            pltpu.VMEM((qpw,), jnp.float32),
            pltpu.VMEM((L * L,), jnp.float32),
            pltpu.SemaphoreType.DMA,
            pltpu.SemaphoreType.DMA,
        ],
        compiler_params=pltpu.CompilerParams(needs_layout_passes=False),
        interpret=interpret,
    )
    def k(tw_hbm, occ_hbm, out_hbm, tw_v, occ_a, occ_b, out_v, tbuf,
          sem_a, sem_b):
        cid = lax.axis_index("c")
        sid = lax.axis_index("s")
        wid = sid * num_cores + cid
        prob = (wid * qpw) // q
        occ_base = wid * (qpw * s)
        pltpu.sync_copy(tw_hbm.at[pl.ds(prob * s, s)], tw_v)
        tws = tuple(tw_v[pl.ds(j * L, L)] for j in range(sv))

        def start_fetch(ci, buf, sem):
            off = occ_base + ci * (chunk_q * s)
            pltpu.make_async_copy(
                occ_hbm.at[pl.ds(off, chunk_q * s)], buf, sem).start()

        def compute_chunk(ci, occ_v):
            def group_body(g, carry2):
                for qq in range(L):
                    qbase = (g * L + qq) * s
                    accs = []
                    for k4 in range(4):
                        a = occ_v[pl.ds(qbase + k4 * L, L)] * tws[k4]
                        for j in range(k4 + 4, sv, 4):
                            a = a + occ_v[pl.ds(qbase + j * L, L)] * tws[j]
                        accs.append(a)
                    acc = (accs[0] + accs[1]) + (accs[2] + accs[3])
                    tbuf[pl.ds(qq * L, L)] = acc
                col = lax.iota(jnp.int32, L) * L
                res = plsc.load_gather(tbuf, [col])
                for c in range(1, L):
                    res = res + plsc.load_gather(tbuf, [col + c])
                out_v[pl.ds(ci * chunk_q + g * L, L)] = res
                return carry2

            lax.fori_loop(0, n_groups, group_body, 0)

        start_fetch(0, occ_a, sem_a)

        def pair_body(p, carry):
            for par, (buf, sem, obuf, osem) in enumerate(
                    ((occ_a, sem_a, occ_b, sem_b),
                     (occ_b, sem_b, occ_a, sem_a))):
                ci = p * 2 + par

                @pl.when(ci + 1 < n_chunks)
                def _():
                    start_fetch(ci + 1, obuf, osem)

                pltpu.make_async_copy(
                    occ_hbm.at[pl.ds(occ_base + ci * (chunk_q * s),
                                     chunk_q * s)], buf, sem).wait()
                compute_chunk(ci, buf)
            return carry

        lax.fori_loop(0, n_chunks // 2, pair_body, 0)
        pltpu.sync_copy(out_v, out_hbm.at[pl.ds(wid * qpw, qpw)])

    return k


def kernel(token_weight_flat, occurrence_flat, valid, symbol_row_splits,
           question_row_splits):
    del symbol_row_splits, question_row_splits
    tw_masked = token_weight_flat * jnp.repeat(valid, S).astype(jnp.float32)
    k = _build(B, Q, S, 2, 16, 64)
    logits = k(tw_masked, occurrence_flat)
    return logits, valid
